# baseline (speedup 1.0000x reference)
import functools

import jax
import jax.numpy as jnp
from jax import lax
from jax.experimental import pallas as pl
from jax.experimental.pallas import tpu as pltpu
from jax.experimental.pallas import tpu_sc as plsc

BATCH = 16384
HALF = BATCH // 2
D = 64
NU = 1_000_000
FOLD = 512_000
NC = 2
NS = 16
LANES = 16
NW = NC * NS
BPW = 512
CPW = 256
FIRE = 16


def _pack_body(lo_ref, hi_ref, out_ref):
    lo = lo_ref[...]
    hi = hi_ref[...]
    out_ref[...] = jnp.concatenate([lo.T, hi.T], axis=1)


def _pack(ut):
    BU = 512
    n_hi_blocks = (NU + BU - 1) // BU - 1
    return pl.pallas_call(
        _pack_body,
        grid=(FOLD // BU,),
        in_specs=[
            pl.BlockSpec((D, BU), lambda i: (0, i)),
            pl.BlockSpec((D, BU),
                         lambda i: (0, jnp.minimum(FOLD // BU + i,
                                                   n_hi_blocks))),
        ],
        out_specs=pl.BlockSpec((BU, 128), lambda i: (i, 0)),
        out_shape=jax.ShapeDtypeStruct((FOLD, 128), jnp.float32),
    )(ut, ut)


def _sc_body(uids_hbm, iids_hbm, up_hbm, iemb_hbm, z_hbm,
             uidx_v, iidx_v, urows_v, irows_v, sem_u, sem_i):
    wid = lax.axis_index("s") * NC + lax.axis_index("c")
    base = wid * CPW
    pltpu.sync_copy(uids_hbm.at[pl.ds(base, CPW)], uidx_v.at[pl.ds(0, CPW)])
    pltpu.sync_copy(uids_hbm.at[pl.ds(HALF + base, CPW)],
                    uidx_v.at[pl.ds(CPW, CPW)])
    pltpu.sync_copy(iids_hbm.at[pl.ds(base, CPW)], iidx_v.at[pl.ds(0, CPW)])
    pltpu.sync_copy(iids_hbm.at[pl.ds(HALF + base, CPW)],
                    iidx_v.at[pl.ds(CPW, CPW)])

    def fire(g, carry):
        r0 = g * FIRE
        uvec = uidx_v[pl.ds(r0, FIRE)]
        ivec = iidx_v[pl.ds(r0, FIRE)]
        rvec = jnp.where(uvec >= FOLD, uvec - FOLD, uvec)
        for j in range(FIRE):
            b = r0 + j
            pltpu.async_copy(up_hbm.at[rvec[j]], urows_v.at[b], sem_u)
            q = g * (FIRE // 2) + j // 2
            dst_c = pl.ds((j % 2) * D, D)
            pltpu.async_copy(iemb_hbm.at[ivec[j]], irows_v.at[q, dst_c],
                             sem_i)
        return carry

    lax.fori_loop(0, BPW // FIRE, fire, 0)
    pltpu.make_async_copy(up_hbm.at[pl.ds(0, BPW)], urows_v, sem_u).wait()
    pltpu.make_async_copy(up_hbm.at[pl.ds(0, CPW)], irows_v, sem_i).wait()

    def mul_a(g, carry):
        r0 = g * FIRE
        uvec = uidx_v[pl.ds(r0, FIRE)]
        hvec = jnp.where(uvec >= FOLD, D, 0)
        for j in range(FIRE):
            b = r0 + j
            q = g * (FIRE // 2) + j // 2
            hj = hvec[j]
            for c in range(D // LANES):
                u = urows_v[b, pl.ds(hj + c * LANES, LANES)]
                iv = irows_v[q, pl.ds((j % 2) * D + c * LANES, LANES)]
                urows_v[b, pl.ds(c * LANES, LANES)] = u * iv
        return carry

    def mul_b(g, carry):
        r0 = g * FIRE
        uvec = uidx_v[pl.ds(CPW + r0, FIRE)]
        hvec = jnp.where(uvec >= FOLD, D, 0)
        for j in range(FIRE):
            b = CPW + r0 + j
            q = (CPW + r0 + j) // 2
            hj = hvec[j]
            for c in range(D // LANES):
                u = urows_v[b, pl.ds(hj + c * LANES, LANES)]
                iv = irows_v[q, pl.ds((j % 2) * D + c * LANES, LANES)]
                urows_v[b - CPW, pl.ds(D + c * LANES, LANES)] = u * iv
        return carry

    lax.fori_loop(0, CPW // FIRE, mul_a, 0)
    lax.fori_loop(0, CPW // FIRE, mul_b, 0)
    pltpu.sync_copy(urows_v.at[pl.ds(0, CPW)], z_hbm.at[pl.ds(base, CPW)])


@functools.partial(
    pl.kernel,
    mesh=plsc.VectorSubcoreMesh(core_axis_name="c", subcore_axis_name="s"),
    out_type=jax.ShapeDtypeStruct((HALF, 128), jnp.float32),
    scratch_types=[
        pltpu.VMEM((BPW,), jnp.int32),
        pltpu.VMEM((BPW,), jnp.int32),
        pltpu.VMEM((BPW, 128), jnp.float32),
        pltpu.VMEM((CPW, 128), jnp.float32),
        pltpu.SemaphoreType.DMA,
        pltpu.SemaphoreType.DMA,
    ],
)
def _sc_gather_mul(uids, iids, up, iemb, z, uidx_v, iidx_v, urows_v,
                   irows_v, sem_u, sem_i):
    _sc_body(uids, iids, up, iemb, z, uidx_v, iidx_v, urows_v, irows_v,
             sem_u, sem_i)


def _mlp_body(z_ref, w0_ref, b0_ref, w1_ref, b1_ref, hw_ref, hb_ref, out_ref):
    z = z_ref[...]
    h = lax.dot_general(z, w0_ref[...], (((1,), (0,)), ((), ())),
                        precision=lax.Precision.HIGHEST,
                        preferred_element_type=jnp.float32)
    h = jnp.maximum(h + b0_ref[...], 0.0)
    h = lax.dot_general(h, w1_ref[...], (((1,), (0,)), ((), ())),
                        precision=lax.Precision.HIGHEST,
                        preferred_element_type=jnp.float32)
    h = jnp.maximum(h + b1_ref[...], 0.0)
    e = h * hw_ref[...]
    s0 = jnp.sum(e[:, :D], axis=1, keepdims=True)
    s1 = jnp.sum(e[:, D:], axis=1, keepdims=True)
    out_ref[...] = jnp.concatenate([s0, s1], axis=1) + hb_ref[0, 0]


def _mlp(z, W0p, b0p, W1p, b1p, hWp, hb):
    BLK = 1024
    return pl.pallas_call(
        _mlp_body,
        grid=(HALF // BLK,),
        in_specs=[
            pl.BlockSpec((BLK, 128), lambda i: (i, 0)),
            pl.BlockSpec((128, 128), lambda i: (0, 0)),
            pl.BlockSpec((1, 128), lambda i: (0, 0)),
            pl.BlockSpec((128, 128), lambda i: (0, 0)),
            pl.BlockSpec((1, 128), lambda i: (0, 0)),
            pl.BlockSpec((1, 128), lambda i: (0, 0)),
            pl.BlockSpec(memory_space=pltpu.SMEM),
        ],
        out_specs=pl.BlockSpec((BLK, 2), lambda i: (i, 0)),
        out_shape=jax.ShapeDtypeStruct((HALF, 2), jnp.float32),
    )(z, W0p, b0p, W1p, b1p, hWp, hb.reshape(1, 1))


def kernel(user_ids, item_ids, user_emb, item_emb, W0, b0, W1, b1, hW, hb):
    uids = user_ids.astype(jnp.int32)
    iids = item_ids.astype(jnp.int32)

    up = _pack(user_emb.T)
    z = _sc_gather_mul(uids, iids, up, item_emb)

    zpad = jnp.zeros((D, D), jnp.float32)
    W0p = jnp.block([[W0.T, zpad], [zpad, W0.T]])
    W1p = jnp.block([[W1.T, zpad], [zpad, W1.T]])
    b0p = jnp.tile(b0.reshape(1, D), (1, 2))
    b1p = jnp.tile(b1.reshape(1, D), (1, 2))
    hWp = jnp.tile(hW.reshape(1, D), (1, 2))

    out2 = _mlp(z, W0p, b0p, W1p, b1p, hWp, hb)
    return jnp.concatenate([out2[:, :1], out2[:, 1:]], axis=0)


# speedup vs baseline: 1.0991x; 1.0991x over previous
import functools

import jax
import jax.numpy as jnp
from jax import lax
from jax.experimental import pallas as pl
from jax.experimental.pallas import tpu as pltpu
from jax.experimental.pallas import tpu_sc as plsc

BATCH = 16384
HALF = BATCH // 2
D = 64
NU = 1_000_000
FOLD = 512_000
NC = 2
NS = 16
LANES = 16
NW = NC * NS
BPW = 512
CPW = 256
FIRE = 16


def _pack_body(lo_ref, hi_ref, out_ref):
    lo = lo_ref[...]
    hi = hi_ref[...]
    eye = jnp.eye(D, dtype=jnp.float32)
    lo_t = lax.dot_general(lo, eye, (((0,), (0,)), ((), ())),
                           precision=lax.Precision.HIGHEST,
                           preferred_element_type=jnp.float32)
    hi_t = lax.dot_general(hi, eye, (((0,), (0,)), ((), ())),
                           precision=lax.Precision.HIGHEST,
                           preferred_element_type=jnp.float32)
    out_ref[...] = jnp.concatenate([lo_t, hi_t], axis=1)


def _pack(ut):
    BU = 1024
    n_hi_blocks = (NU + BU - 1) // BU - 1
    return pl.pallas_call(
        _pack_body,
        grid=(FOLD // BU,),
        in_specs=[
            pl.BlockSpec((D, BU), lambda i: (0, i)),
            pl.BlockSpec((D, BU),
                         lambda i: (0, jnp.minimum(FOLD // BU + i,
                                                   n_hi_blocks))),
        ],
        out_specs=pl.BlockSpec((BU, 128), lambda i: (i, 0)),
        out_shape=jax.ShapeDtypeStruct((FOLD, 128), jnp.float32),
    )(ut, ut)


def _sc_body(uids_hbm, iids_hbm, up_hbm, iemb_hbm, z_hbm,
             uidx_v, iidx_v, urows_v, irows_v, sem_u, sem_i):
    wid = lax.axis_index("s") * NC + lax.axis_index("c")
    base = wid * CPW
    pltpu.sync_copy(uids_hbm.at[pl.ds(base, CPW)], uidx_v.at[pl.ds(0, CPW)])
    pltpu.sync_copy(uids_hbm.at[pl.ds(HALF + base, CPW)],
                    uidx_v.at[pl.ds(CPW, CPW)])
    pltpu.sync_copy(iids_hbm.at[pl.ds(base, CPW)], iidx_v.at[pl.ds(0, CPW)])
    pltpu.sync_copy(iids_hbm.at[pl.ds(HALF + base, CPW)],
                    iidx_v.at[pl.ds(CPW, CPW)])

    def fire(g, carry):
        r0 = g * FIRE
        uvec = uidx_v[pl.ds(r0, FIRE)]
        ivec = iidx_v[pl.ds(r0, FIRE)]
        rvec = jnp.where(uvec >= FOLD, uvec - FOLD, uvec)
        for j in range(FIRE):
            b = r0 + j
            pltpu.async_copy(up_hbm.at[rvec[j]], urows_v.at[b], sem_u)
            q = g * (FIRE // 2) + j // 2
            dst_c = pl.ds((j % 2) * D, D)
            pltpu.async_copy(iemb_hbm.at[ivec[j]], irows_v.at[q, dst_c],
                             sem_i)
        return carry

    lax.fori_loop(0, BPW // FIRE, fire, 0)
    pltpu.make_async_copy(up_hbm.at[pl.ds(0, BPW)], urows_v, sem_u).wait()
    pltpu.make_async_copy(up_hbm.at[pl.ds(0, CPW)], irows_v, sem_i).wait()

    def mul_a(g, carry):
        r0 = g * FIRE
        uvec = uidx_v[pl.ds(r0, FIRE)]
        hvec = jnp.where(uvec >= FOLD, D, 0)
        for j in range(FIRE):
            b = r0 + j
            q = g * (FIRE // 2) + j // 2
            hj = hvec[j]
            for c in range(D // LANES):
                u = urows_v[b, pl.ds(hj + c * LANES, LANES)]
                iv = irows_v[q, pl.ds((j % 2) * D + c * LANES, LANES)]
                urows_v[b, pl.ds(c * LANES, LANES)] = u * iv
        return carry

    def mul_b(g, carry):
        r0 = g * FIRE
        uvec = uidx_v[pl.ds(CPW + r0, FIRE)]
        hvec = jnp.where(uvec >= FOLD, D, 0)
        for j in range(FIRE):
            b = CPW + r0 + j
            q = (CPW + r0 + j) // 2
            hj = hvec[j]
            for c in range(D // LANES):
                u = urows_v[b, pl.ds(hj + c * LANES, LANES)]
                iv = irows_v[q, pl.ds((j % 2) * D + c * LANES, LANES)]
                urows_v[b - CPW, pl.ds(D + c * LANES, LANES)] = u * iv
        return carry

    lax.fori_loop(0, CPW // FIRE, mul_a, 0)
    lax.fori_loop(0, CPW // FIRE, mul_b, 0)
    pltpu.sync_copy(urows_v.at[pl.ds(0, CPW)], z_hbm.at[pl.ds(base, CPW)])


@functools.partial(
    pl.kernel,
    mesh=plsc.VectorSubcoreMesh(core_axis_name="c", subcore_axis_name="s"),
    out_type=jax.ShapeDtypeStruct((HALF, 128), jnp.float32),
    scratch_types=[
        pltpu.VMEM((BPW,), jnp.int32),
        pltpu.VMEM((BPW,), jnp.int32),
        pltpu.VMEM((BPW, 128), jnp.float32),
        pltpu.VMEM((CPW, 128), jnp.float32),
        pltpu.SemaphoreType.DMA,
        pltpu.SemaphoreType.DMA,
    ],
)
def _sc_gather_mul(uids, iids, up, iemb, z, uidx_v, iidx_v, urows_v,
                   irows_v, sem_u, sem_i):
    _sc_body(uids, iids, up, iemb, z, uidx_v, iidx_v, urows_v, irows_v,
             sem_u, sem_i)


def _mlp_body(z_ref, w0_ref, b0_ref, w1_ref, b1_ref, hw_ref, hb_ref, out_ref):
    z = z_ref[...]
    h = lax.dot_general(z, w0_ref[...], (((1,), (0,)), ((), ())),
                        precision=lax.Precision.HIGHEST,
                        preferred_element_type=jnp.float32)
    h = jnp.maximum(h + b0_ref[...], 0.0)
    h = lax.dot_general(h, w1_ref[...], (((1,), (0,)), ((), ())),
                        precision=lax.Precision.HIGHEST,
                        preferred_element_type=jnp.float32)
    h = jnp.maximum(h + b1_ref[...], 0.0)
    e = h * hw_ref[...]
    s0 = jnp.sum(e[:, :D], axis=1, keepdims=True)
    s1 = jnp.sum(e[:, D:], axis=1, keepdims=True)
    out_ref[...] = jnp.concatenate([s0, s1], axis=1) + hb_ref[0, 0]


def _mlp(z, W0p, b0p, W1p, b1p, hWp, hb):
    BLK = 1024
    return pl.pallas_call(
        _mlp_body,
        grid=(HALF // BLK,),
        in_specs=[
            pl.BlockSpec((BLK, 128), lambda i: (i, 0)),
            pl.BlockSpec((128, 128), lambda i: (0, 0)),
            pl.BlockSpec((1, 128), lambda i: (0, 0)),
            pl.BlockSpec((128, 128), lambda i: (0, 0)),
            pl.BlockSpec((1, 128), lambda i: (0, 0)),
            pl.BlockSpec((1, 128), lambda i: (0, 0)),
            pl.BlockSpec(memory_space=pltpu.SMEM),
        ],
        out_specs=pl.BlockSpec((BLK, 2), lambda i: (i, 0)),
        out_shape=jax.ShapeDtypeStruct((HALF, 2), jnp.float32),
    )(z, W0p, b0p, W1p, b1p, hWp, hb.reshape(1, 1))


def kernel(user_ids, item_ids, user_emb, item_emb, W0, b0, W1, b1, hW, hb):
    uids = user_ids.astype(jnp.int32)
    iids = item_ids.astype(jnp.int32)

    up = _pack(user_emb.T)
    z = _sc_gather_mul(uids, iids, up, item_emb)

    zpad = jnp.zeros((D, D), jnp.float32)
    W0p = jnp.block([[W0.T, zpad], [zpad, W0.T]])
    W1p = jnp.block([[W1.T, zpad], [zpad, W1.T]])
    b0p = jnp.tile(b0.reshape(1, D), (1, 2))
    b1p = jnp.tile(b1.reshape(1, D), (1, 2))
    hWp = jnp.tile(hW.reshape(1, D), (1, 2))

    out2 = _mlp(z, W0p, b0p, W1p, b1p, hWp, hb)
    return jnp.concatenate([out2[:, :1], out2[:, 1:]], axis=0)


# speedup vs baseline: 1.1547x; 1.0505x over previous
import functools

import jax
import jax.numpy as jnp
from jax import lax
from jax.experimental import pallas as pl
from jax.experimental.pallas import tpu as pltpu
from jax.experimental.pallas import tpu_sc as plsc

BATCH = 16384
HALF = BATCH // 2
D = 64
NU = 1_000_000
FOLD = 512_000
NC = 2
NS = 16
LANES = 16
NW = NC * NS
BPW = 512
CPW = 256
FIRE = 16


def _pack_body(lo_ref, hi_ref, out_ref):
    lo = lo_ref[...]
    hi = hi_ref[...]
    eye = jnp.eye(D, dtype=jnp.float32)
    lo_t = lax.dot_general(lo, eye, (((0,), (0,)), ((), ())),
                           precision=lax.Precision.HIGHEST,
                           preferred_element_type=jnp.float32)
    hi_t = lax.dot_general(hi, eye, (((0,), (0,)), ((), ())),
                           precision=lax.Precision.HIGHEST,
                           preferred_element_type=jnp.float32)
    out_ref[...] = jnp.concatenate([lo_t, hi_t], axis=1)


def _pack(ut):
    BU = 1024
    n_hi_blocks = (NU + BU - 1) // BU - 1
    return pl.pallas_call(
        _pack_body,
        grid=(FOLD // BU,),
        in_specs=[
            pl.BlockSpec((D, BU), lambda i: (0, i)),
            pl.BlockSpec((D, BU),
                         lambda i: (0, jnp.minimum(FOLD // BU + i,
                                                   n_hi_blocks))),
        ],
        out_specs=pl.BlockSpec((BU, 128), lambda i: (i, 0)),
        out_shape=jax.ShapeDtypeStruct((FOLD, 128), jnp.float32),
    )(ut, ut)


def _sc_body(uids_hbm, iids_hbm, up_hbm, iemb_hbm, z_hbm,
             uidx_v, iidx_v, urows_v, irows_v, sem_u, sem_i):
    wid = lax.axis_index("s") * NC + lax.axis_index("c")
    base = wid * CPW
    pltpu.sync_copy(uids_hbm.at[pl.ds(base, CPW)], uidx_v.at[pl.ds(0, CPW)])
    pltpu.sync_copy(uids_hbm.at[pl.ds(HALF + base, CPW)],
                    uidx_v.at[pl.ds(CPW, CPW)])
    pltpu.sync_copy(iids_hbm.at[pl.ds(base, CPW)], iidx_v.at[pl.ds(0, CPW)])
    pltpu.sync_copy(iids_hbm.at[pl.ds(HALF + base, CPW)],
                    iidx_v.at[pl.ds(CPW, CPW)])

    def fire(g, carry):
        r0 = g * FIRE
        uvec = uidx_v[pl.ds(r0, FIRE)]
        ivec = iidx_v[pl.ds(r0, FIRE)]
        rvec = uvec >> 1
        for j in range(FIRE):
            b = r0 + j
            pltpu.async_copy(up_hbm.at[rvec[j]], urows_v.at[b], sem_u)
            q = g * (FIRE // 2) + j // 2
            dst_c = pl.ds((j % 2) * D, D)
            pltpu.async_copy(iemb_hbm.at[ivec[j]], irows_v.at[q, dst_c],
                             sem_i)
        return carry

    lax.fori_loop(0, BPW // FIRE, fire, 0)
    pltpu.make_async_copy(up_hbm.at[pl.ds(0, BPW)], urows_v, sem_u).wait()
    pltpu.make_async_copy(up_hbm.at[pl.ds(0, CPW)], irows_v, sem_i).wait()

    def mul_a(g, carry):
        r0 = g * FIRE
        uvec = uidx_v[pl.ds(r0, FIRE)]
        hvec = (uvec & 1) * D
        for j in range(FIRE):
            b = r0 + j
            q = g * (FIRE // 2) + j // 2
            hj = hvec[j]
            for c in range(D // LANES):
                u = urows_v[b, pl.ds(hj + c * LANES, LANES)]
                iv = irows_v[q, pl.ds((j % 2) * D + c * LANES, LANES)]
                urows_v[b, pl.ds(c * LANES, LANES)] = u * iv
        return carry

    def mul_b(g, carry):
        r0 = g * FIRE
        uvec = uidx_v[pl.ds(CPW + r0, FIRE)]
        hvec = (uvec & 1) * D
        for j in range(FIRE):
            b = CPW + r0 + j
            q = (CPW + r0 + j) // 2
            hj = hvec[j]
            for c in range(D // LANES):
                u = urows_v[b, pl.ds(hj + c * LANES, LANES)]
                iv = irows_v[q, pl.ds((j % 2) * D + c * LANES, LANES)]
                urows_v[b - CPW, pl.ds(D + c * LANES, LANES)] = u * iv
        return carry

    lax.fori_loop(0, CPW // FIRE, mul_a, 0)
    lax.fori_loop(0, CPW // FIRE, mul_b, 0)
    pltpu.sync_copy(urows_v.at[pl.ds(0, CPW)], z_hbm.at[pl.ds(base, CPW)])


@functools.partial(
    pl.kernel,
    mesh=plsc.VectorSubcoreMesh(core_axis_name="c", subcore_axis_name="s"),
    out_type=jax.ShapeDtypeStruct((HALF, 128), jnp.float32),
    scratch_types=[
        pltpu.VMEM((BPW,), jnp.int32),
        pltpu.VMEM((BPW,), jnp.int32),
        pltpu.VMEM((BPW, 128), jnp.float32),
        pltpu.VMEM((CPW, 128), jnp.float32),
        pltpu.SemaphoreType.DMA,
        pltpu.SemaphoreType.DMA,
    ],
)
def _sc_gather_mul(uids, iids, up, iemb, z, uidx_v, iidx_v, urows_v,
                   irows_v, sem_u, sem_i):
    _sc_body(uids, iids, up, iemb, z, uidx_v, iidx_v, urows_v, irows_v,
             sem_u, sem_i)


def _mlp_body(z_ref, w0_ref, b0_ref, w1_ref, b1_ref, hw_ref, hb_ref, out_ref):
    z = z_ref[...]
    h = lax.dot_general(z, w0_ref[...], (((1,), (0,)), ((), ())),
                        precision=lax.Precision.HIGHEST,
                        preferred_element_type=jnp.float32)
    h = jnp.maximum(h + b0_ref[...], 0.0)
    h = lax.dot_general(h, w1_ref[...], (((1,), (0,)), ((), ())),
                        precision=lax.Precision.HIGHEST,
                        preferred_element_type=jnp.float32)
    h = jnp.maximum(h + b1_ref[...], 0.0)
    e = h * hw_ref[...]
    s0 = jnp.sum(e[:, :D], axis=1, keepdims=True)
    s1 = jnp.sum(e[:, D:], axis=1, keepdims=True)
    out_ref[...] = jnp.concatenate([s0, s1], axis=1) + hb_ref[0, 0]


def _mlp(z, W0p, b0p, W1p, b1p, hWp, hb):
    BLK = 1024
    return pl.pallas_call(
        _mlp_body,
        grid=(HALF // BLK,),
        in_specs=[
            pl.BlockSpec((BLK, 128), lambda i: (i, 0)),
            pl.BlockSpec((128, 128), lambda i: (0, 0)),
            pl.BlockSpec((1, 128), lambda i: (0, 0)),
            pl.BlockSpec((128, 128), lambda i: (0, 0)),
            pl.BlockSpec((1, 128), lambda i: (0, 0)),
            pl.BlockSpec((1, 128), lambda i: (0, 0)),
            pl.BlockSpec(memory_space=pltpu.SMEM),
        ],
        out_specs=pl.BlockSpec((BLK, 2), lambda i: (i, 0)),
        out_shape=jax.ShapeDtypeStruct((HALF, 2), jnp.float32),
    )(z, W0p, b0p, W1p, b1p, hWp, hb.reshape(1, 1))


def kernel(user_ids, item_ids, user_emb, item_emb, W0, b0, W1, b1, hW, hb):
    uids = user_ids.astype(jnp.int32)
    iids = item_ids.astype(jnp.int32)

    up = user_emb.reshape(NU // 2, 128)
    z = _sc_gather_mul(uids, iids, up, item_emb)

    zpad = jnp.zeros((D, D), jnp.float32)
    W0p = jnp.block([[W0.T, zpad], [zpad, W0.T]])
    W1p = jnp.block([[W1.T, zpad], [zpad, W1.T]])
    b0p = jnp.tile(b0.reshape(1, D), (1, 2))
    b1p = jnp.tile(b1.reshape(1, D), (1, 2))
    hWp = jnp.tile(hW.reshape(1, D), (1, 2))

    out2 = _mlp(z, W0p, b0p, W1p, b1p, hWp, hb)
    return jnp.concatenate([out2[:, :1], out2[:, 1:]], axis=0)


# speedup vs baseline: 1.3763x; 1.1919x over previous
import functools

import jax
import jax.numpy as jnp
from jax import lax
from jax.experimental import pallas as pl
from jax.experimental.pallas import tpu as pltpu
from jax.experimental.pallas import tpu_sc as plsc

BATCH = 16384
HALF = BATCH // 2
D = 64
NU = 1_000_000
FOLD = 512_000
NC = 2
NS = 16
LANES = 16
NW = NC * NS
BPW = 512
CPW = 256
FIRE = 16


def _pack_body(lo_ref, hi_ref, out_ref):
    lo = lo_ref[...]
    hi = hi_ref[...]
    eye = jnp.eye(D, dtype=jnp.float32)
    lo_t = lax.dot_general(lo, eye, (((0,), (0,)), ((), ())),
                           precision=lax.Precision.HIGHEST,
                           preferred_element_type=jnp.float32)
    hi_t = lax.dot_general(hi, eye, (((0,), (0,)), ((), ())),
                           precision=lax.Precision.HIGHEST,
                           preferred_element_type=jnp.float32)
    out_ref[...] = jnp.concatenate([lo_t, hi_t], axis=1)


def _pack(ut):
    BU = 10240
    n_hi_blocks = (NU + BU - 1) // BU - 1
    return pl.pallas_call(
        _pack_body,
        grid=(FOLD // BU,),
        in_specs=[
            pl.BlockSpec((D, BU), lambda i: (0, i)),
            pl.BlockSpec((D, BU),
                         lambda i: (0, jnp.minimum(FOLD // BU + i,
                                                   n_hi_blocks))),
        ],
        out_specs=pl.BlockSpec((BU, 128), lambda i: (i, 0)),
        out_shape=jax.ShapeDtypeStruct((FOLD, 128), jnp.float32),
    )(ut, ut)


def _sc_body(uids_hbm, iids_hbm, up_hbm, iemb_hbm, z_hbm,
             uidx_v, iidx_v, urows_v, irows_v, sem_u, sem_i):
    wid = lax.axis_index("s") * NC + lax.axis_index("c")
    base = wid * CPW
    pltpu.sync_copy(uids_hbm.at[pl.ds(base, CPW)], uidx_v.at[pl.ds(0, CPW)])
    pltpu.sync_copy(uids_hbm.at[pl.ds(HALF + base, CPW)],
                    uidx_v.at[pl.ds(CPW, CPW)])
    pltpu.sync_copy(iids_hbm.at[pl.ds(base, CPW)], iidx_v.at[pl.ds(0, CPW)])
    pltpu.sync_copy(iids_hbm.at[pl.ds(HALF + base, CPW)],
                    iidx_v.at[pl.ds(CPW, CPW)])

    def fire(g, carry):
        r0 = g * FIRE
        uvec = uidx_v[pl.ds(r0, FIRE)]
        ivec = iidx_v[pl.ds(r0, FIRE)]
        rvec = jnp.where(uvec >= FOLD, uvec - FOLD, uvec)
        for j in range(FIRE):
            b = r0 + j
            pltpu.async_copy(up_hbm.at[rvec[j]], urows_v.at[b], sem_u)
            q = g * (FIRE // 2) + j // 2
            dst_c = pl.ds((j % 2) * D, D)
            pltpu.async_copy(iemb_hbm.at[ivec[j]], irows_v.at[q, dst_c],
                             sem_i)
        return carry

    lax.fori_loop(0, BPW // FIRE, fire, 0)
    pltpu.make_async_copy(up_hbm.at[pl.ds(0, BPW)], urows_v, sem_u).wait()
    pltpu.make_async_copy(up_hbm.at[pl.ds(0, CPW)], irows_v, sem_i).wait()

    def mul_a(g, carry):
        r0 = g * FIRE
        uvec = uidx_v[pl.ds(r0, FIRE)]
        hvec = jnp.where(uvec >= FOLD, D, 0)
        for j in range(FIRE):
            b = r0 + j
            q = g * (FIRE // 2) + j // 2
            hj = hvec[j]
            for c in range(D // LANES):
                u = urows_v[b, pl.ds(hj + c * LANES, LANES)]
                iv = irows_v[q, pl.ds((j % 2) * D + c * LANES, LANES)]
                urows_v[b, pl.ds(c * LANES, LANES)] = u * iv
        return carry

    def mul_b(g, carry):
        r0 = g * FIRE
        uvec = uidx_v[pl.ds(CPW + r0, FIRE)]
        hvec = jnp.where(uvec >= FOLD, D, 0)
        for j in range(FIRE):
            b = CPW + r0 + j
            q = (CPW + r0 + j) // 2
            hj = hvec[j]
            for c in range(D // LANES):
                u = urows_v[b, pl.ds(hj + c * LANES, LANES)]
                iv = irows_v[q, pl.ds((j % 2) * D + c * LANES, LANES)]
                urows_v[b - CPW, pl.ds(D + c * LANES, LANES)] = u * iv
        return carry

    lax.fori_loop(0, CPW // FIRE, mul_a, 0)
    lax.fori_loop(0, CPW // FIRE, mul_b, 0)
    pltpu.sync_copy(urows_v.at[pl.ds(0, CPW)], z_hbm.at[pl.ds(base, CPW)])


@functools.partial(
    pl.kernel,
    mesh=plsc.VectorSubcoreMesh(core_axis_name="c", subcore_axis_name="s"),
    out_type=jax.ShapeDtypeStruct((HALF, 128), jnp.float32),
    scratch_types=[
        pltpu.VMEM((BPW,), jnp.int32),
        pltpu.VMEM((BPW,), jnp.int32),
        pltpu.VMEM((BPW, 128), jnp.float32),
        pltpu.VMEM((CPW, 128), jnp.float32),
        pltpu.SemaphoreType.DMA,
        pltpu.SemaphoreType.DMA,
    ],
)
def _sc_gather_mul(uids, iids, up, iemb, z, uidx_v, iidx_v, urows_v,
                   irows_v, sem_u, sem_i):
    _sc_body(uids, iids, up, iemb, z, uidx_v, iidx_v, urows_v, irows_v,
             sem_u, sem_i)


def _mlp_body(z_ref, w0_ref, b0_ref, w1_ref, b1_ref, hw_ref, hb_ref, out_ref):
    z = z_ref[...]
    h = lax.dot_general(z, w0_ref[...], (((1,), (0,)), ((), ())),
                        precision=lax.Precision.HIGHEST,
                        preferred_element_type=jnp.float32)
    h = jnp.maximum(h + b0_ref[...], 0.0)
    h = lax.dot_general(h, w1_ref[...], (((1,), (0,)), ((), ())),
                        precision=lax.Precision.HIGHEST,
                        preferred_element_type=jnp.float32)
    h = jnp.maximum(h + b1_ref[...], 0.0)
    e = h * hw_ref[...]
    s0 = jnp.sum(e[:, :D], axis=1, keepdims=True)
    s1 = jnp.sum(e[:, D:], axis=1, keepdims=True)
    out_ref[...] = jnp.concatenate([s0, s1], axis=1) + hb_ref[0, 0]


def _mlp(z, W0p, b0p, W1p, b1p, hWp, hb):
    BLK = 1024
    return pl.pallas_call(
        _mlp_body,
        grid=(HALF // BLK,),
        in_specs=[
            pl.BlockSpec((BLK, 128), lambda i: (i, 0)),
            pl.BlockSpec((128, 128), lambda i: (0, 0)),
            pl.BlockSpec((1, 128), lambda i: (0, 0)),
            pl.BlockSpec((128, 128), lambda i: (0, 0)),
            pl.BlockSpec((1, 128), lambda i: (0, 0)),
            pl.BlockSpec((1, 128), lambda i: (0, 0)),
            pl.BlockSpec(memory_space=pltpu.SMEM),
        ],
        out_specs=pl.BlockSpec((BLK, 2), lambda i: (i, 0)),
        out_shape=jax.ShapeDtypeStruct((HALF, 2), jnp.float32),
    )(z, W0p, b0p, W1p, b1p, hWp, hb.reshape(1, 1))


def kernel(user_ids, item_ids, user_emb, item_emb, W0, b0, W1, b1, hW, hb):
    uids = user_ids.astype(jnp.int32)
    iids = item_ids.astype(jnp.int32)

    up = _pack(user_emb.T)
    z = _sc_gather_mul(uids, iids, up, item_emb)

    zpad = jnp.zeros((D, D), jnp.float32)
    W0p = jnp.block([[W0.T, zpad], [zpad, W0.T]])
    W1p = jnp.block([[W1.T, zpad], [zpad, W1.T]])
    b0p = jnp.tile(b0.reshape(1, D), (1, 2))
    b1p = jnp.tile(b1.reshape(1, D), (1, 2))
    hWp = jnp.tile(hW.reshape(1, D), (1, 2))

    out2 = _mlp(z, W0p, b0p, W1p, b1p, hWp, hb)
    return jnp.concatenate([out2[:, :1], out2[:, 1:]], axis=0)


# speedup vs baseline: 2.6201x; 1.9037x over previous
import functools

import jax
import jax.numpy as jnp
from jax import lax
from jax.experimental import pallas as pl
from jax.experimental.pallas import tpu as pltpu
from jax.experimental.pallas import tpu_sc as plsc

BATCH = 16384
HALF = BATCH // 2
D = 64
NU = 1_000_000
FOLD = 512_000
NC = 2
NS = 16
LANES = 16
NW = NC * NS
BPW = 512
CPW = 256
FIRE = 16


def _pack_body(lo_ref, hi_ref, out_ref):
    lo = lo_ref[...]
    hi = hi_ref[...]
    eye = jnp.eye(D, dtype=jnp.float32)
    lo_t = lax.dot_general(lo, eye, (((0,), (0,)), ((), ())),
                           precision=lax.Precision.DEFAULT,
                           preferred_element_type=jnp.float32)
    hi_t = lax.dot_general(hi, eye, (((0,), (0,)), ((), ())),
                           precision=lax.Precision.DEFAULT,
                           preferred_element_type=jnp.float32)
    out_ref[...] = jnp.concatenate([lo_t, hi_t], axis=1)


def _pack(ut):
    BU = 10240
    n_hi_blocks = (NU + BU - 1) // BU - 1
    return pl.pallas_call(
        _pack_body,
        grid=(FOLD // BU,),
        in_specs=[
            pl.BlockSpec((D, BU), lambda i: (0, i)),
            pl.BlockSpec((D, BU),
                         lambda i: (0, jnp.minimum(FOLD // BU + i,
                                                   n_hi_blocks))),
        ],
        out_specs=pl.BlockSpec((BU, 128), lambda i: (i, 0)),
        out_shape=jax.ShapeDtypeStruct((FOLD, 128), jnp.float32),
    )(ut, ut)


def _sc_body(uids_hbm, iids_hbm, up_hbm, iemb_hbm, z_hbm,
             uidx_v, iidx_v, urows_v, irows_v, sem_u, sem_i):
    wid = lax.axis_index("s") * NC + lax.axis_index("c")
    base = wid * CPW
    pltpu.sync_copy(uids_hbm.at[pl.ds(base, CPW)], uidx_v.at[pl.ds(0, CPW)])
    pltpu.sync_copy(uids_hbm.at[pl.ds(HALF + base, CPW)],
                    uidx_v.at[pl.ds(CPW, CPW)])
    pltpu.sync_copy(iids_hbm.at[pl.ds(base, CPW)], iidx_v.at[pl.ds(0, CPW)])
    pltpu.sync_copy(iids_hbm.at[pl.ds(HALF + base, CPW)],
                    iidx_v.at[pl.ds(CPW, CPW)])

    def fire(g, carry):
        r0 = g * FIRE
        uvec = uidx_v[pl.ds(r0, FIRE)]
        ivec = iidx_v[pl.ds(r0, FIRE)]
        rvec = jnp.where(uvec >= FOLD, uvec - FOLD, uvec)
        for j in range(FIRE):
            b = r0 + j
            pltpu.async_copy(up_hbm.at[rvec[j]], urows_v.at[b], sem_u)
            q = g * (FIRE // 2) + j // 2
            dst_c = pl.ds((j % 2) * D, D)
            pltpu.async_copy(iemb_hbm.at[ivec[j]], irows_v.at[q, dst_c],
                             sem_i)
        return carry

    lax.fori_loop(0, BPW // FIRE, fire, 0)
    pltpu.make_async_copy(up_hbm.at[pl.ds(0, BPW)], urows_v, sem_u).wait()
    pltpu.make_async_copy(up_hbm.at[pl.ds(0, CPW)], irows_v, sem_i).wait()

    def mul_a(g, carry):
        r0 = g * FIRE
        uvec = uidx_v[pl.ds(r0, FIRE)]
        hvec = jnp.where(uvec >= FOLD, D, 0)
        for j in range(FIRE):
            b = r0 + j
            q = g * (FIRE // 2) + j // 2
            hj = hvec[j]
            for c in range(D // LANES):
                u = urows_v[b, pl.ds(hj + c * LANES, LANES)]
                iv = irows_v[q, pl.ds((j % 2) * D + c * LANES, LANES)]
                urows_v[b, pl.ds(c * LANES, LANES)] = u * iv
        return carry

    def mul_b(g, carry):
        r0 = g * FIRE
        uvec = uidx_v[pl.ds(CPW + r0, FIRE)]
        hvec = jnp.where(uvec >= FOLD, D, 0)
        for j in range(FIRE):
            b = CPW + r0 + j
            q = (CPW + r0 + j) // 2
            hj = hvec[j]
            for c in range(D // LANES):
                u = urows_v[b, pl.ds(hj + c * LANES, LANES)]
                iv = irows_v[q, pl.ds((j % 2) * D + c * LANES, LANES)]
                urows_v[b - CPW, pl.ds(D + c * LANES, LANES)] = u * iv
        return carry

    lax.fori_loop(0, CPW // FIRE, mul_a, 0)
    lax.fori_loop(0, CPW // FIRE, mul_b, 0)
    pltpu.sync_copy(urows_v.at[pl.ds(0, CPW)], z_hbm.at[pl.ds(base, CPW)])


@functools.partial(
    pl.kernel,
    mesh=plsc.VectorSubcoreMesh(core_axis_name="c", subcore_axis_name="s"),
    out_type=jax.ShapeDtypeStruct((HALF, 128), jnp.float32),
    scratch_types=[
        pltpu.VMEM((BPW,), jnp.int32),
        pltpu.VMEM((BPW,), jnp.int32),
        pltpu.VMEM((BPW, 128), jnp.float32),
        pltpu.VMEM((CPW, 128), jnp.float32),
        pltpu.SemaphoreType.DMA,
        pltpu.SemaphoreType.DMA,
    ],
)
def _sc_gather_mul(uids, iids, up, iemb, z, uidx_v, iidx_v, urows_v,
                   irows_v, sem_u, sem_i):
    _sc_body(uids, iids, up, iemb, z, uidx_v, iidx_v, urows_v, irows_v,
             sem_u, sem_i)


def _mlp_body(z_ref, w0_ref, b0_ref, w1_ref, b1_ref, hw_ref, hb_ref, out_ref):
    z = z_ref[...]
    h = lax.dot_general(z, w0_ref[...], (((1,), (0,)), ((), ())),
                        precision=lax.Precision.HIGHEST,
                        preferred_element_type=jnp.float32)
    h = jnp.maximum(h + b0_ref[...], 0.0)
    h = lax.dot_general(h, w1_ref[...], (((1,), (0,)), ((), ())),
                        precision=lax.Precision.HIGHEST,
                        preferred_element_type=jnp.float32)
    h = jnp.maximum(h + b1_ref[...], 0.0)
    e = h * hw_ref[...]
    s0 = jnp.sum(e[:, :D], axis=1, keepdims=True)
    s1 = jnp.sum(e[:, D:], axis=1, keepdims=True)
    out_ref[...] = jnp.concatenate([s0, s1], axis=1) + hb_ref[0, 0]


def _mlp(z, W0p, b0p, W1p, b1p, hWp, hb):
    BLK = 1024
    return pl.pallas_call(
        _mlp_body,
        grid=(HALF // BLK,),
        in_specs=[
            pl.BlockSpec((BLK, 128), lambda i: (i, 0)),
            pl.BlockSpec((128, 128), lambda i: (0, 0)),
            pl.BlockSpec((1, 128), lambda i: (0, 0)),
            pl.BlockSpec((128, 128), lambda i: (0, 0)),
            pl.BlockSpec((1, 128), lambda i: (0, 0)),
            pl.BlockSpec((1, 128), lambda i: (0, 0)),
            pl.BlockSpec(memory_space=pltpu.SMEM),
        ],
        out_specs=pl.BlockSpec((BLK, 2), lambda i: (i, 0)),
        out_shape=jax.ShapeDtypeStruct((HALF, 2), jnp.float32),
    )(z, W0p, b0p, W1p, b1p, hWp, hb.reshape(1, 1))


def kernel(user_ids, item_ids, user_emb, item_emb, W0, b0, W1, b1, hW, hb):
    uids = user_ids.astype(jnp.int32)
    iids = item_ids.astype(jnp.int32)

    up = _pack(user_emb.T)
    z = _sc_gather_mul(uids, iids, up, item_emb)

    zpad = jnp.zeros((D, D), jnp.float32)
    W0p = jnp.block([[W0.T, zpad], [zpad, W0.T]])
    W1p = jnp.block([[W1.T, zpad], [zpad, W1.T]])
    b0p = jnp.tile(b0.reshape(1, D), (1, 2))
    b1p = jnp.tile(b1.reshape(1, D), (1, 2))
    hWp = jnp.tile(hW.reshape(1, D), (1, 2))

    out2 = _mlp(z, W0p, b0p, W1p, b1p, hWp, hb)
    return jnp.concatenate([out2[:, :1], out2[:, 1:]], axis=0)


# speedup vs baseline: 2.7179x; 1.0373x over previous
import functools

import jax
import jax.numpy as jnp
from jax import lax
from jax.experimental import pallas as pl
from jax.experimental.pallas import tpu as pltpu
from jax.experimental.pallas import tpu_sc as plsc

BATCH = 16384
HALF = BATCH // 2
D = 64
NU = 1_000_000
FOLD = 512_000
NC = 2
NS = 16
LANES = 16
NW = NC * NS
BPW = 512
CPW = 256
FIRE = 16


def _pack_body(lo_ref, hi_ref, out_ref):
    lo = lo_ref[...]
    hi = hi_ref[...]
    eye = jnp.eye(D, dtype=jnp.float32)
    lo_t = lax.dot_general(lo, eye, (((0,), (0,)), ((), ())),
                           precision=lax.Precision.DEFAULT,
                           preferred_element_type=jnp.float32)
    hi_t = lax.dot_general(hi, eye, (((0,), (0,)), ((), ())),
                           precision=lax.Precision.DEFAULT,
                           preferred_element_type=jnp.float32)
    out_ref[...] = jnp.concatenate([lo_t, hi_t], axis=1)


def _pack(ut):
    BU = 10240
    n_hi_blocks = (NU + BU - 1) // BU - 1
    return pl.pallas_call(
        _pack_body,
        grid=(FOLD // BU,),
        in_specs=[
            pl.BlockSpec((D, BU), lambda i: (0, i)),
            pl.BlockSpec((D, BU),
                         lambda i: (0, jnp.minimum(FOLD // BU + i,
                                                   n_hi_blocks))),
        ],
        out_specs=pl.BlockSpec((BU, 128), lambda i: (i, 0)),
        out_shape=jax.ShapeDtypeStruct((FOLD, 128), jnp.float32),
    )(ut, ut)


def _sc_body(uids_hbm, iids_hbm, up_hbm, iemb_hbm, z_hbm,
             uidx_v, iidx_v, urows_v, irows_v, sem_u, sem_i):
    wid = lax.axis_index("s") * NC + lax.axis_index("c")
    base = wid * CPW
    pltpu.sync_copy(uids_hbm.at[pl.ds(base, CPW)], uidx_v.at[pl.ds(0, CPW)])
    pltpu.sync_copy(uids_hbm.at[pl.ds(HALF + base, CPW)],
                    uidx_v.at[pl.ds(CPW, CPW)])
    pltpu.sync_copy(iids_hbm.at[pl.ds(base, CPW)], iidx_v.at[pl.ds(0, CPW)])
    pltpu.sync_copy(iids_hbm.at[pl.ds(HALF + base, CPW)],
                    iidx_v.at[pl.ds(CPW, CPW)])

    def fire(g, carry):
        r0 = g * FIRE
        uvec = uidx_v[pl.ds(r0, FIRE)]
        ivec = iidx_v[pl.ds(r0, FIRE)]
        rvec = jnp.where(uvec >= FOLD, uvec - FOLD, uvec)
        for j in range(FIRE):
            b = r0 + j
            pltpu.async_copy(up_hbm.at[rvec[j]], urows_v.at[b], sem_u)
            q = g * (FIRE // 2) + j // 2
            dst_c = pl.ds((j % 2) * D, D)
            pltpu.async_copy(iemb_hbm.at[ivec[j]], irows_v.at[q, dst_c],
                             sem_i)
        return carry

    lax.fori_loop(0, BPW // FIRE, fire, 0)
    pltpu.make_async_copy(up_hbm.at[pl.ds(0, BPW)], urows_v, sem_u).wait()
    pltpu.make_async_copy(up_hbm.at[pl.ds(0, CPW)], irows_v, sem_i).wait()

    def mul_a(g, carry):
        r0 = g * FIRE
        uvec = uidx_v[pl.ds(r0, FIRE)]
        hvec = jnp.where(uvec >= FOLD, D, 0)
        for j in range(FIRE):
            b = r0 + j
            q = g * (FIRE // 2) + j // 2
            hj = hvec[j]
            for c in range(D // LANES):
                u = urows_v[b, pl.ds(hj + c * LANES, LANES)]
                iv = irows_v[q, pl.ds((j % 2) * D + c * LANES, LANES)]
                urows_v[b, pl.ds(c * LANES, LANES)] = u * iv
        return carry

    def mul_b(g, carry):
        r0 = g * FIRE
        uvec = uidx_v[pl.ds(CPW + r0, FIRE)]
        hvec = jnp.where(uvec >= FOLD, D, 0)
        for j in range(FIRE):
            b = CPW + r0 + j
            q = (CPW + r0 + j) // 2
            hj = hvec[j]
            for c in range(D // LANES):
                u = urows_v[b, pl.ds(hj + c * LANES, LANES)]
                iv = irows_v[q, pl.ds((j % 2) * D + c * LANES, LANES)]
                urows_v[b - CPW, pl.ds(D + c * LANES, LANES)] = u * iv
        return carry

    lax.fori_loop(0, CPW // FIRE, mul_a, 0)
    lax.fori_loop(0, CPW // FIRE, mul_b, 0)
    pltpu.sync_copy(urows_v.at[pl.ds(0, CPW)], z_hbm.at[pl.ds(base, CPW)])


@functools.partial(
    pl.kernel,
    mesh=plsc.VectorSubcoreMesh(core_axis_name="c", subcore_axis_name="s"),
    out_type=jax.ShapeDtypeStruct((HALF, 128), jnp.float32),
    scratch_types=[
        pltpu.VMEM((BPW,), jnp.int32),
        pltpu.VMEM((BPW,), jnp.int32),
        pltpu.VMEM((BPW, 128), jnp.float32),
        pltpu.VMEM((CPW, 128), jnp.float32),
        pltpu.SemaphoreType.DMA,
        pltpu.SemaphoreType.DMA,
    ],
)
def _sc_gather_mul(uids, iids, up, iemb, z, uidx_v, iidx_v, urows_v,
                   irows_v, sem_u, sem_i):
    _sc_body(uids, iids, up, iemb, z, uidx_v, iidx_v, urows_v, irows_v,
             sem_u, sem_i)


def _mlp_body(z_ref, w0_ref, b0_ref, w1_ref, b1_ref, hw_ref, hb_ref, out_ref):
    z = z_ref[...]
    h = lax.dot_general(z, w0_ref[...], (((1,), (0,)), ((), ())),
                        precision=lax.Precision.DEFAULT,
                        preferred_element_type=jnp.float32)
    h = jnp.maximum(h + b0_ref[...], 0.0)
    h = lax.dot_general(h, w1_ref[...], (((1,), (0,)), ((), ())),
                        precision=lax.Precision.DEFAULT,
                        preferred_element_type=jnp.float32)
    h = jnp.maximum(h + b1_ref[...], 0.0)
    e = h * hw_ref[...]
    s0 = jnp.sum(e[:, :D], axis=1, keepdims=True)
    s1 = jnp.sum(e[:, D:], axis=1, keepdims=True)
    out_ref[...] = jnp.concatenate([s0, s1], axis=1) + hb_ref[0, 0]


def _mlp(z, W0p, b0p, W1p, b1p, hWp, hb):
    BLK = 1024
    return pl.pallas_call(
        _mlp_body,
        grid=(HALF // BLK,),
        in_specs=[
            pl.BlockSpec((BLK, 128), lambda i: (i, 0)),
            pl.BlockSpec((128, 128), lambda i: (0, 0)),
            pl.BlockSpec((1, 128), lambda i: (0, 0)),
            pl.BlockSpec((128, 128), lambda i: (0, 0)),
            pl.BlockSpec((1, 128), lambda i: (0, 0)),
            pl.BlockSpec((1, 128), lambda i: (0, 0)),
            pl.BlockSpec(memory_space=pltpu.SMEM),
        ],
        out_specs=pl.BlockSpec((BLK, 2), lambda i: (i, 0)),
        out_shape=jax.ShapeDtypeStruct((HALF, 2), jnp.float32),
    )(z, W0p, b0p, W1p, b1p, hWp, hb.reshape(1, 1))


def kernel(user_ids, item_ids, user_emb, item_emb, W0, b0, W1, b1, hW, hb):
    uids = user_ids.astype(jnp.int32)
    iids = item_ids.astype(jnp.int32)

    up = _pack(user_emb.T)
    z = _sc_gather_mul(uids, iids, up, item_emb)

    zpad = jnp.zeros((D, D), jnp.float32)
    W0p = jnp.block([[W0.T, zpad], [zpad, W0.T]])
    W1p = jnp.block([[W1.T, zpad], [zpad, W1.T]])
    b0p = jnp.tile(b0.reshape(1, D), (1, 2))
    b1p = jnp.tile(b1.reshape(1, D), (1, 2))
    hWp = jnp.tile(hW.reshape(1, D), (1, 2))

    out2 = _mlp(z, W0p, b0p, W1p, b1p, hWp, hb)
    return jnp.concatenate([out2[:, :1], out2[:, 1:]], axis=0)
